# baseline (speedup 1.0000x reference)
import functools

import jax
import jax.numpy as jnp
from jax import lax
from jax.experimental import pallas as pl
from jax.experimental.pallas import tpu as pltpu
from jax.experimental.pallas import tpu_sc as plsc

_NUM_BINS = 5
_TB = 2.5
_LN2 = 0.6931471805599453

_NC = 2
_NS = 16
_NW = _NC * _NS
_LANES = 16

_N = 4194304
_PER_W = _N // _NW
_CH = 16384
_CHUNKS = _PER_W // _CH


def _vlog(t):
  bits = plsc.bitcast(t, jnp.int32)
  e = ((bits >> 23) - 127).astype(jnp.float32)
  m = plsc.bitcast((bits & 0x007FFFFF) | 0x3F800000, jnp.float32)
  s = (m - 1.0) / (m + 1.0)
  s2 = s * s
  p = 2.0 / 7.0
  p = 2.0 / 5.0 + s2 * p
  p = 2.0 / 3.0 + s2 * p
  return e * _LN2 + s * (2.0 + s2 * p)


def _lane_shift(v, idx):
  return v.at[idx].get(mode="promise_in_bounds")


def _sc_body(x_hbm, p_hbm, z_hbm, lj_hbm, pbuf, t_xk, t_rw, t_yk, t_dy, t_s8,
             t_mid, t_dk8, t_h1, t_a1, xbuf0, xbuf1, zbuf0, zbuf1, ljbuf0,
             ljbuf1, sem_in0, sem_in1, sem_out0, sem_out1):
  wid = lax.axis_index("s") * _NC + lax.axis_index("c")
  base = wid * _PER_W
  xbufs = (xbuf0, xbuf1)
  zbufs = (zbuf0, zbuf1)
  ljbufs = (ljbuf0, ljbuf1)
  sems_in = (sem_in0, sem_in1)
  sems_out = (sem_out0, sem_out1)

  in_d = [None, None]
  in_d[0] = pltpu.async_copy(x_hbm.at[pl.ds(base, _CH)], xbufs[0],
                             sems_in[0])

  pltpu.sync_copy(p_hbm, pbuf)
  pv = pbuf[...]
  io = lax.iota(jnp.int32, 16)
  mask_w = io < _NUM_BINS
  mask_h = (io >= _NUM_BINS) & (io < 2 * _NUM_BINS)
  neg = jnp.float32(-3.4e38)

  mw = jnp.max(jnp.where(mask_w, pv, neg))
  ew = jnp.exp(pv - mw)
  sw = jnp.sum(jnp.where(mask_w, ew, 0.0))
  w_v = (ew * (2.0 * _TB)) / sw
  mh = jnp.max(jnp.where(mask_h, pv, neg))
  eh = jnp.exp(pv - mh)
  sh = jnp.sum(jnp.where(mask_h, eh, 0.0))
  h_v = (eh * (2.0 * _TB)) / sh
  d_v = jnp.maximum(pv, 0.0) + _vlog(1.0 + jnp.exp(-jnp.abs(pv))) + 1e-5

  cw = plsc.cumsum(jnp.where(mask_w, w_v, 0.0))
  ch = plsc.cumsum(jnp.where(mask_h, h_v, 0.0))

  cap = jnp.int32(15)
  x_k1 = cw - _TB
  x_k = jnp.where(io == 0, -_TB,
                  _lane_shift(cw, jnp.maximum(io - 1, 0)) - _TB)
  rw = 1.0 / (x_k1 - x_k + 1e-8)
  y_k = jnp.where(io == 0, -_TB,
                  _lane_shift(ch, jnp.minimum(io + 4, cap)) - _TB)
  y_k1 = _lane_shift(ch, jnp.minimum(io + 5, cap)) - _TB
  dy = y_k1 - y_k
  d_k = _lane_shift(d_v, jnp.minimum(io + 10, cap))
  d_k1 = _lane_shift(d_v, jnp.minimum(io + 11, cap))
  s_k = _lane_shift(h_v, jnp.minimum(io + 5, cap)) / w_v
  s8 = s_k + 1e-8
  mid = d_k + d_k1 - 2.0 * s_k
  dk8 = d_k + 1e-8
  h1 = s8 - d_k
  a1 = 2.0 * h1

  t_xk[...] = x_k
  t_rw[...] = rw
  t_yk[...] = y_k
  t_dy[...] = dy
  t_s8[...] = s8
  t_mid[...] = mid
  t_dk8[...] = dk8
  t_h1[...] = h1
  t_a1[...] = a1

  k1 = jnp.sum(jnp.where(io == 0, x_k1, 0.0))
  k2 = jnp.sum(jnp.where(io == 1, x_k1, 0.0))
  k3 = jnp.sum(jnp.where(io == 2, x_k1, 0.0))
  k4 = jnp.sum(jnp.where(io == 3, x_k1, 0.0))

  def compute(xb, zb, ljb):
    @plsc.parallel_loop(0, _CH, step=_LANES, unroll=8)
    def _loop(off):
      sl = pl.ds(off, _LANES)
      xv = xb[sl]
      inside = jnp.abs(xv) <= _TB
      b = ((k1 < xv).astype(jnp.int32) + (k2 < xv).astype(jnp.int32)
           + (k3 < xv).astype(jnp.int32) + (k4 < xv).astype(jnp.int32))
      g_xk = plsc.load_gather(t_xk, [b])
      g_rw = plsc.load_gather(t_rw, [b])
      g_yk = plsc.load_gather(t_yk, [b])
      g_dy = plsc.load_gather(t_dy, [b])
      g_s8 = plsc.load_gather(t_s8, [b])
      g_mid = plsc.load_gather(t_mid, [b])
      g_dk8 = plsc.load_gather(t_dk8, [b])
      g_h1 = plsc.load_gather(t_h1, [b])
      g_a1 = plsc.load_gather(t_a1, [b])

      xi = (xv - g_xk) * g_rw
      t = xi * (1.0 - xi)
      d8 = g_s8 + g_mid * t
      inv = 1.0 / d8
      numz = xi * (g_dk8 + g_h1 * xi)
      z_in = g_yk + g_dy * (numz * inv)
      numj = (g_mid * xi + g_a1) * xi + g_dk8
      r = g_s8 * inv
      lj_in = _vlog(numj * (r * r))
      zb[sl] = jnp.where(inside, z_in, xv)
      ljb[sl] = jnp.where(inside, lj_in, 0.0)

  out_d = [None, None]
  for g in range(_CHUNKS):
    b = g % 2
    off = base + g * _CH
    in_d[b].wait()
    if g + 1 < _CHUNKS:
      nb = (g + 1) % 2
      in_d[nb] = pltpu.async_copy(x_hbm.at[pl.ds(off + _CH, _CH)],
                                  xbufs[nb], sems_in[nb])
    if out_d[b] is not None:
      out_d[b][0].wait()
      out_d[b][1].wait()
    compute(xbufs[b], zbufs[b], ljbufs[b])
    out_d[b] = (
        pltpu.async_copy(zbufs[b], z_hbm.at[pl.ds(off, _CH)], sems_out[b]),
        pltpu.async_copy(ljbufs[b], lj_hbm.at[pl.ds(off, _CH)],
                         sems_out[b]),
    )
  out_d[0][0].wait()
  out_d[0][1].wait()
  out_d[1][0].wait()
  out_d[1][1].wait()


@jax.jit
def _run(x_flat, params):
  mesh = plsc.VectorSubcoreMesh(core_axis_name="c", subcore_axis_name="s",
                                num_cores=_NC, num_subcores=_NS)
  f = pl.kernel(
      _sc_body,
      out_type=[jax.ShapeDtypeStruct((_N,), jnp.float32),
                jax.ShapeDtypeStruct((_N,), jnp.float32)],
      mesh=mesh,
      compiler_params=pltpu.CompilerParams(needs_layout_passes=False),
      scratch_types=[
          pltpu.VMEM((16,), jnp.float32),
          pltpu.VMEM((16,), jnp.float32),
          pltpu.VMEM((16,), jnp.float32),
          pltpu.VMEM((16,), jnp.float32),
          pltpu.VMEM((16,), jnp.float32),
          pltpu.VMEM((16,), jnp.float32),
          pltpu.VMEM((16,), jnp.float32),
          pltpu.VMEM((16,), jnp.float32),
          pltpu.VMEM((16,), jnp.float32),
          pltpu.VMEM((16,), jnp.float32),
          pltpu.VMEM((_CH,), jnp.float32),
          pltpu.VMEM((_CH,), jnp.float32),
          pltpu.VMEM((_CH,), jnp.float32),
          pltpu.VMEM((_CH,), jnp.float32),
          pltpu.VMEM((_CH,), jnp.float32),
          pltpu.VMEM((_CH,), jnp.float32),
          pltpu.SemaphoreType.DMA,
          pltpu.SemaphoreType.DMA,
          pltpu.SemaphoreType.DMA,
          pltpu.SemaphoreType.DMA,
      ],
  )
  return f(x_flat, params)


def kernel(x, params):
  z, lj = _run(x[:, 0], params)
  return (z[:, None], lj)


# speedup vs baseline: 1.4898x; 1.4898x over previous
import functools

import jax
import jax.numpy as jnp
from jax import lax
from jax.experimental import pallas as pl
from jax.experimental.pallas import tpu as pltpu
from jax.experimental.pallas import tpu_sc as plsc

_NUM_BINS = 5
_TB = 2.5
_LN2 = 0.6931471805599453

_NC = 2
_NS = 16
_NW = _NC * _NS
_LANES = 16

_N = 4194304
_PER_W = _N // _NW
_CH = 16384
_CHUNKS = _PER_W // _CH


def _vlog(t):
  bits = plsc.bitcast(t, jnp.int32)
  e = ((bits >> 23) - 127).astype(jnp.float32)
  m = plsc.bitcast((bits & 0x007FFFFF) | 0x3F800000, jnp.float32)
  s = (m - 1.0) / (m + 1.0)
  s2 = s * s
  p = 2.0 / 7.0
  p = 2.0 / 5.0 + s2 * p
  p = 2.0 / 3.0 + s2 * p
  return e * _LN2 + s * (2.0 + s2 * p)


def _lane_shift(v, idx):
  return v.at[idx].get(mode="promise_in_bounds")


def _sc_body(x_hbm, p_hbm, z_hbm, lj_hbm, pbuf, t_xk, t_rw, t_yk, t_dy, t_s8,
             t_mid, t_dk8, t_h1, t_a1, xbuf0, xbuf1, zbuf0, zbuf1, ljbuf0,
             ljbuf1, sem_in0, sem_in1, sem_out0, sem_out1):
  wid = lax.axis_index("s") * _NC + lax.axis_index("c")
  base = wid * _PER_W
  xbufs = (xbuf0, xbuf1)
  zbufs = (zbuf0, zbuf1)
  ljbufs = (ljbuf0, ljbuf1)
  sems_in = (sem_in0, sem_in1)
  sems_out = (sem_out0, sem_out1)

  in_d = [None, None]
  in_d[0] = pltpu.async_copy(x_hbm.at[pl.ds(base, _CH)], xbufs[0],
                             sems_in[0])

  pltpu.sync_copy(p_hbm, pbuf)
  pv = pbuf[...]
  io = lax.iota(jnp.int32, 16)
  mask_w = io < _NUM_BINS
  mask_h = (io >= _NUM_BINS) & (io < 2 * _NUM_BINS)
  neg = jnp.float32(-3.4e38)

  mw = jnp.max(jnp.where(mask_w, pv, neg))
  ew = jnp.exp(pv - mw)
  sw = jnp.sum(jnp.where(mask_w, ew, 0.0))
  w_v = (ew * (2.0 * _TB)) / sw
  mh = jnp.max(jnp.where(mask_h, pv, neg))
  eh = jnp.exp(pv - mh)
  sh = jnp.sum(jnp.where(mask_h, eh, 0.0))
  h_v = (eh * (2.0 * _TB)) / sh
  d_v = jnp.maximum(pv, 0.0) + _vlog(1.0 + jnp.exp(-jnp.abs(pv))) + 1e-5

  cw = plsc.cumsum(jnp.where(mask_w, w_v, 0.0))
  ch = plsc.cumsum(jnp.where(mask_h, h_v, 0.0))

  cap = jnp.int32(15)
  x_k1 = cw - _TB
  x_k = jnp.where(io == 0, -_TB,
                  _lane_shift(cw, jnp.maximum(io - 1, 0)) - _TB)
  rw = 1.0 / (x_k1 - x_k + 1e-8)
  y_k = jnp.where(io == 0, -_TB,
                  _lane_shift(ch, jnp.minimum(io + 4, cap)) - _TB)
  y_k1 = _lane_shift(ch, jnp.minimum(io + 5, cap)) - _TB
  dy = y_k1 - y_k
  d_k = _lane_shift(d_v, jnp.minimum(io + 10, cap))
  d_k1 = _lane_shift(d_v, jnp.minimum(io + 11, cap))
  s_k = _lane_shift(h_v, jnp.minimum(io + 5, cap)) / w_v
  s8 = s_k + 1e-8
  mid = d_k + d_k1 - 2.0 * s_k
  dk8 = d_k + 1e-8
  h1 = s8 - d_k
  a1 = 2.0 * h1

  t_xk[...] = x_k
  t_rw[...] = rw
  t_yk[...] = y_k
  t_dy[...] = dy
  t_s8[...] = s8
  t_mid[...] = mid
  t_dk8[...] = dk8
  t_h1[...] = h1
  t_a1[...] = a1

  k1 = jnp.sum(jnp.where(io == 0, x_k1, 0.0))
  k2 = jnp.sum(jnp.where(io == 1, x_k1, 0.0))
  k3 = jnp.sum(jnp.where(io == 2, x_k1, 0.0))
  k4 = jnp.sum(jnp.where(io == 3, x_k1, 0.0))

  def compute(xb, zb, ljb):
    @plsc.parallel_loop(0, _CH, step=_LANES, unroll=4)
    def _loop(off):
      sl = pl.ds(off, _LANES)
      xv = xb[sl]
      inside = jnp.abs(xv) <= _TB
      b = ((k1 < xv).astype(jnp.int32) + (k2 < xv).astype(jnp.int32)
           + (k3 < xv).astype(jnp.int32) + (k4 < xv).astype(jnp.int32))
      g_xk = plsc.load_gather(t_xk, [b])
      g_rw = plsc.load_gather(t_rw, [b])
      g_yk = plsc.load_gather(t_yk, [b])
      g_dy = plsc.load_gather(t_dy, [b])
      g_s8 = plsc.load_gather(t_s8, [b])
      g_mid = plsc.load_gather(t_mid, [b])
      g_dk8 = plsc.load_gather(t_dk8, [b])
      g_h1 = plsc.load_gather(t_h1, [b])
      g_a1 = plsc.load_gather(t_a1, [b])

      xi = (xv - g_xk) * g_rw
      t = xi * (1.0 - xi)
      d8 = g_s8 + g_mid * t
      inv = 1.0 / d8
      numz = xi * (g_dk8 + g_h1 * xi)
      z_in = g_yk + g_dy * (numz * inv)
      numj = (g_mid * xi + g_a1) * xi + g_dk8
      r = g_s8 * inv
      lj_in = _vlog(numj * (r * r))
      zb[sl] = jnp.where(inside, z_in, xv)
      ljb[sl] = jnp.where(inside, lj_in, 0.0)

  out_d = [None, None]
  for g in range(_CHUNKS):
    b = g % 2
    off = base + g * _CH
    in_d[b].wait()
    if g + 1 < _CHUNKS:
      nb = (g + 1) % 2
      in_d[nb] = pltpu.async_copy(x_hbm.at[pl.ds(off + _CH, _CH)],
                                  xbufs[nb], sems_in[nb])
    if out_d[b] is not None:
      out_d[b][0].wait()
      out_d[b][1].wait()
    compute(xbufs[b], zbufs[b], ljbufs[b])
    out_d[b] = (
        pltpu.async_copy(zbufs[b], z_hbm.at[pl.ds(off, _CH)], sems_out[b]),
        pltpu.async_copy(ljbufs[b], lj_hbm.at[pl.ds(off, _CH)],
                         sems_out[b]),
    )
  out_d[0][0].wait()
  out_d[0][1].wait()
  out_d[1][0].wait()
  out_d[1][1].wait()


@jax.jit
def _run(x_flat, params):
  mesh = plsc.VectorSubcoreMesh(core_axis_name="c", subcore_axis_name="s",
                                num_cores=_NC, num_subcores=_NS)
  f = pl.kernel(
      _sc_body,
      out_type=[jax.ShapeDtypeStruct((_N,), jnp.float32),
                jax.ShapeDtypeStruct((_N,), jnp.float32)],
      mesh=mesh,
      compiler_params=pltpu.CompilerParams(needs_layout_passes=False),
      scratch_types=[
          pltpu.VMEM((16,), jnp.float32),
          pltpu.VMEM((16,), jnp.float32),
          pltpu.VMEM((16,), jnp.float32),
          pltpu.VMEM((16,), jnp.float32),
          pltpu.VMEM((16,), jnp.float32),
          pltpu.VMEM((16,), jnp.float32),
          pltpu.VMEM((16,), jnp.float32),
          pltpu.VMEM((16,), jnp.float32),
          pltpu.VMEM((16,), jnp.float32),
          pltpu.VMEM((16,), jnp.float32),
          pltpu.VMEM((_CH,), jnp.float32),
          pltpu.VMEM((_CH,), jnp.float32),
          pltpu.VMEM((_CH,), jnp.float32),
          pltpu.VMEM((_CH,), jnp.float32),
          pltpu.VMEM((_CH,), jnp.float32),
          pltpu.VMEM((_CH,), jnp.float32),
          pltpu.SemaphoreType.DMA,
          pltpu.SemaphoreType.DMA,
          pltpu.SemaphoreType.DMA,
          pltpu.SemaphoreType.DMA,
      ],
  )
  return f(x_flat, params)


def kernel(x, params):
  z, lj = _run(x[:, 0], params)
  return (z[:, None], lj)


# speedup vs baseline: 1.6562x; 1.1117x over previous
import functools

import jax
import jax.numpy as jnp
from jax import lax
from jax.experimental import pallas as pl
from jax.experimental.pallas import tpu as pltpu
from jax.experimental.pallas import tpu_sc as plsc

_NUM_BINS = 5
_TB = 2.5
_LN2 = 0.6931471805599453
_MAGIC = 0x3F3504F3
_C1 = 1.9999695786510276
_C3 = 0.6769402206514328

_NC = 2
_NS = 16
_NW = _NC * _NS
_LANES = 16

_N = 4194304
_PER_W = _N // _NW
_CH = 16384
_CHUNKS = _PER_W // _CH

_NCELL = 64
_CELL_SCALE = _NCELL / (2.0 * _TB)


def _vlog(t):
  bits = plsc.bitcast(t, jnp.int32)
  e = (bits - _MAGIC) >> 23
  m = plsc.bitcast(bits - (e << 23), jnp.float32)
  s = (m - 1.0) / (m + 1.0)
  return e.astype(jnp.float32) * _LN2 + s * (_C1 + _C3 * (s * s))


def _lane_shift(v, idx):
  return v.at[idx].get(mode="promise_in_bounds")


def _sc_body(x_hbm, p_hbm, z_hbm, lj_hbm, pbuf, t_q2, t_q1, t_q0, t_p2, t_p1,
             t_p0, t_g2, t_g1, t_g0, t_khi, lut, xbuf0, xbuf1, zbuf0, zbuf1,
             ljbuf0, ljbuf1, sem_in0, sem_in1, sem_out0, sem_out1):
  wid = lax.axis_index("s") * _NC + lax.axis_index("c")
  base = wid * _PER_W
  xbufs = (xbuf0, xbuf1)
  zbufs = (zbuf0, zbuf1)
  ljbufs = (ljbuf0, ljbuf1)
  sems_in = (sem_in0, sem_in1)
  sems_out = (sem_out0, sem_out1)

  in_d = [None, None]
  in_d[0] = pltpu.async_copy(x_hbm.at[pl.ds(base, _CH)], xbufs[0],
                             sems_in[0])

  pltpu.sync_copy(p_hbm, pbuf)
  pv = pbuf[...]
  io = lax.iota(jnp.int32, 16)
  mask_w = io < _NUM_BINS
  mask_h = (io >= _NUM_BINS) & (io < 2 * _NUM_BINS)
  neg = jnp.float32(-3.4e38)

  mw = jnp.max(jnp.where(mask_w, pv, neg))
  ew = jnp.exp(pv - mw)
  sw = jnp.sum(jnp.where(mask_w, ew, 0.0))
  w_v = (ew * (2.0 * _TB)) / sw
  mh = jnp.max(jnp.where(mask_h, pv, neg))
  eh = jnp.exp(pv - mh)
  sh = jnp.sum(jnp.where(mask_h, eh, 0.0))
  h_v = (eh * (2.0 * _TB)) / sh
  d_v = jnp.maximum(pv, 0.0) + _vlog(1.0 + jnp.exp(-jnp.abs(pv))) + 1e-5

  cw = plsc.cumsum(jnp.where(mask_w, w_v, 0.0))
  ch = plsc.cumsum(jnp.where(mask_h, h_v, 0.0))

  cap = jnp.int32(15)
  x_k1 = cw - _TB
  x_k = jnp.where(io == 0, -_TB,
                  _lane_shift(cw, jnp.maximum(io - 1, 0)) - _TB)
  rw = 1.0 / (x_k1 - x_k + 1e-8)
  y_k = jnp.where(io == 0, -_TB,
                  _lane_shift(ch, jnp.minimum(io + 4, cap)) - _TB)
  y_k1 = _lane_shift(ch, jnp.minimum(io + 5, cap)) - _TB
  dy = y_k1 - y_k
  d_k = _lane_shift(d_v, jnp.minimum(io + 10, cap))
  d_k1 = _lane_shift(d_v, jnp.minimum(io + 11, cap))
  s_k = _lane_shift(h_v, jnp.minimum(io + 5, cap)) / w_v
  s8 = s_k + 1e-8
  mid = d_k + d_k1 - 2.0 * s_k
  dk8 = d_k + 1e-8
  h1 = s8 - d_k
  a1 = 2.0 * h1

  u = rw
  v = -rw * x_k
  u2 = u * u
  uv2 = 2.0 * u * v
  v2 = v * v
  q2 = -(mid * u2)
  q1 = mid * u - mid * uv2
  q0 = mid * v - mid * v2 + s8
  a2c = h1 * u2
  a1c = h1 * uv2 + dk8 * u
  a0c = h1 * v2 + dk8 * v
  s82 = s8 * s8
  t_q2[...] = q2
  t_q1[...] = q1
  t_q0[...] = q0
  t_p2[...] = y_k * q2 + dy * a2c
  t_p1[...] = y_k * q1 + dy * a1c
  t_p0[...] = y_k * q0 + dy * a0c
  t_g2[...] = (mid * u2) * s82
  t_g1[...] = (mid * uv2 + a1 * u) * s82
  t_g0[...] = (mid * v2 + a1 * v + dk8) * s82
  t_khi[...] = jnp.where(io >= 4, jnp.float32(3.4e38), x_k1)

  k1 = jnp.sum(jnp.where(io == 0, x_k1, 0.0))
  k2 = jnp.sum(jnp.where(io == 1, x_k1, 0.0))
  k3 = jnp.sum(jnp.where(io == 2, x_k1, 0.0))
  k4 = jnp.sum(jnp.where(io == 3, x_k1, 0.0))
  iof = io.astype(jnp.float32)
  for j in range(_NCELL // 16):
    lo = (iof + (16.0 * j)) * (1.0 / _CELL_SCALE) - _TB
    bj = (jnp.where(k1 < lo, 1, 0) + jnp.where(k2 < lo, 1, 0)
          + jnp.where(k3 < lo, 1, 0) + jnp.where(k4 < lo, 1, 0))
    lut[pl.ds(16 * j, 16)] = bj

  def compute(xb, zb, ljb):
    @plsc.parallel_loop(0, _CH, step=_LANES, unroll=4)
    def _loop(off):
      sl = pl.ds(off, _LANES)
      xv = xb[sl]
      inside = jnp.abs(xv) <= _TB
      uf = xv * _CELL_SCALE + (0.5 * _NCELL)
      uf = jnp.minimum(jnp.maximum(uf, 0.0), _NCELL - 1.0)
      ui = uf.astype(jnp.int32)
      b0 = plsc.load_gather(lut, [ui])
      khi = plsc.load_gather(t_khi, [b0])
      b = b0 + jnp.where(khi < xv, 1, 0)
      g_q2 = plsc.load_gather(t_q2, [b])
      g_q1 = plsc.load_gather(t_q1, [b])
      g_q0 = plsc.load_gather(t_q0, [b])
      g_p2 = plsc.load_gather(t_p2, [b])
      g_p1 = plsc.load_gather(t_p1, [b])
      g_p0 = plsc.load_gather(t_p0, [b])
      g_g2 = plsc.load_gather(t_g2, [b])
      g_g1 = plsc.load_gather(t_g1, [b])
      g_g0 = plsc.load_gather(t_g0, [b])

      qx = (g_q2 * xv + g_q1) * xv + g_q0
      px = (g_p2 * xv + g_p1) * xv + g_p0
      gx = (g_g2 * xv + g_g1) * xv + g_g0
      inv = 1.0 / qx
      z_in = px * inv
      lj_in = _vlog(gx * (inv * inv))
      zb[sl] = jnp.where(inside, z_in, xv)
      ljb[sl] = jnp.where(inside, lj_in, 0.0)

  out_d = [None, None]
  for g in range(_CHUNKS):
    b = g % 2
    off = base + g * _CH
    in_d[b].wait()
    if g + 1 < _CHUNKS:
      nb = (g + 1) % 2
      in_d[nb] = pltpu.async_copy(x_hbm.at[pl.ds(off + _CH, _CH)],
                                  xbufs[nb], sems_in[nb])
    if out_d[b] is not None:
      out_d[b][0].wait()
      out_d[b][1].wait()
    compute(xbufs[b], zbufs[b], ljbufs[b])
    out_d[b] = (
        pltpu.async_copy(zbufs[b], z_hbm.at[pl.ds(off, _CH)], sems_out[b]),
        pltpu.async_copy(ljbufs[b], lj_hbm.at[pl.ds(off, _CH)],
                         sems_out[b]),
    )
  out_d[0][0].wait()
  out_d[0][1].wait()
  out_d[1][0].wait()
  out_d[1][1].wait()


@jax.jit
def _run(x_flat, params):
  mesh = plsc.VectorSubcoreMesh(core_axis_name="c", subcore_axis_name="s",
                                num_cores=_NC, num_subcores=_NS)
  f = pl.kernel(
      _sc_body,
      out_type=[jax.ShapeDtypeStruct((_N,), jnp.float32),
                jax.ShapeDtypeStruct((_N,), jnp.float32)],
      mesh=mesh,
      compiler_params=pltpu.CompilerParams(needs_layout_passes=False),
      scratch_types=[
          pltpu.VMEM((16,), jnp.float32),
          pltpu.VMEM((16,), jnp.float32),
          pltpu.VMEM((16,), jnp.float32),
          pltpu.VMEM((16,), jnp.float32),
          pltpu.VMEM((16,), jnp.float32),
          pltpu.VMEM((16,), jnp.float32),
          pltpu.VMEM((16,), jnp.float32),
          pltpu.VMEM((16,), jnp.float32),
          pltpu.VMEM((16,), jnp.float32),
          pltpu.VMEM((16,), jnp.float32),
          pltpu.VMEM((16,), jnp.float32),
          pltpu.VMEM((_NCELL,), jnp.int32),
          pltpu.VMEM((_CH,), jnp.float32),
          pltpu.VMEM((_CH,), jnp.float32),
          pltpu.VMEM((_CH,), jnp.float32),
          pltpu.VMEM((_CH,), jnp.float32),
          pltpu.VMEM((_CH,), jnp.float32),
          pltpu.VMEM((_CH,), jnp.float32),
          pltpu.SemaphoreType.DMA,
          pltpu.SemaphoreType.DMA,
          pltpu.SemaphoreType.DMA,
          pltpu.SemaphoreType.DMA,
      ],
  )
  return f(x_flat, params)


def kernel(x, params):
  z, lj = _run(x[:, 0], params)
  return (z[:, None], lj)


# speedup vs baseline: 1.7603x; 1.0628x over previous
import functools

import jax
import jax.numpy as jnp
from jax import lax
from jax.experimental import pallas as pl
from jax.experimental.pallas import tpu as pltpu
from jax.experimental.pallas import tpu_sc as plsc

_NUM_BINS = 5
_TB = 2.5
_LN2 = 0.6931471805599453
_MAGIC = 0x3F3504F3
_C1 = 1.9999695786510276
_C3 = 0.6769402206514328

_NC = 2
_NS = 16
_NW = _NC * _NS
_LANES = 16

_N = 4194304
_PER_W = _N // _NW
_CH = 16384
_CHUNKS = _PER_W // _CH

_NCELL = 64
_CELL_SCALE = _NCELL / (2.0 * _TB)
_NLUT = 80


def _vlog(t):
  bits = plsc.bitcast(t, jnp.int32)
  e = (bits - _MAGIC) >> 23
  m = plsc.bitcast(bits - (e << 23), jnp.float32)
  s = (m - 1.0) / (m + 1.0)
  return e.astype(jnp.float32) * _LN2 + s * (_C1 + _C3 * (s * s))


def _lane_shift(v, idx):
  return v.at[idx].get(mode="promise_in_bounds")


def _sc_body(x_hbm, p_hbm, z_hbm, lj_hbm, pbuf, t_q2, t_q1, t_q0, t_p2, t_p1,
             t_p0, t_g2, t_g1, t_g0, t_khi, lut, xbuf0, xbuf1, zbuf0, zbuf1,
             ljbuf0, ljbuf1, sem_in0, sem_in1, sem_out0, sem_out1):
  wid = lax.axis_index("s") * _NC + lax.axis_index("c")
  base = wid * _PER_W
  xbufs = (xbuf0, xbuf1)
  zbufs = (zbuf0, zbuf1)
  ljbufs = (ljbuf0, ljbuf1)
  sems_in = (sem_in0, sem_in1)
  sems_out = (sem_out0, sem_out1)

  in_d = [None, None]
  in_d[0] = pltpu.async_copy(x_hbm.at[pl.ds(base, _CH)], xbufs[0],
                             sems_in[0])

  pltpu.sync_copy(p_hbm, pbuf)
  pv = pbuf[...]
  io = lax.iota(jnp.int32, 16)
  mask_w = io < _NUM_BINS
  mask_h = (io >= _NUM_BINS) & (io < 2 * _NUM_BINS)
  neg = jnp.float32(-3.4e38)

  mw = jnp.max(jnp.where(mask_w, pv, neg))
  ew = jnp.exp(pv - mw)
  sw = jnp.sum(jnp.where(mask_w, ew, 0.0))
  w_v = (ew * (2.0 * _TB)) / sw
  mh = jnp.max(jnp.where(mask_h, pv, neg))
  eh = jnp.exp(pv - mh)
  sh = jnp.sum(jnp.where(mask_h, eh, 0.0))
  h_v = (eh * (2.0 * _TB)) / sh
  d_v = jnp.maximum(pv, 0.0) + _vlog(1.0 + jnp.exp(-jnp.abs(pv))) + 1e-5

  cw = plsc.cumsum(jnp.where(mask_w, w_v, 0.0))
  ch = plsc.cumsum(jnp.where(mask_h, h_v, 0.0))

  cap = jnp.int32(15)
  x_k1 = cw - _TB
  x_k = jnp.where(io == 0, -_TB,
                  _lane_shift(cw, jnp.maximum(io - 1, 0)) - _TB)
  rw = 1.0 / (x_k1 - x_k + 1e-8)
  y_k = jnp.where(io == 0, -_TB,
                  _lane_shift(ch, jnp.minimum(io + 4, cap)) - _TB)
  y_k1 = _lane_shift(ch, jnp.minimum(io + 5, cap)) - _TB
  dy = y_k1 - y_k
  d_k = _lane_shift(d_v, jnp.minimum(io + 10, cap))
  d_k1 = _lane_shift(d_v, jnp.minimum(io + 11, cap))
  s_k = _lane_shift(h_v, jnp.minimum(io + 5, cap)) / w_v
  s8 = s_k + 1e-8
  mid = d_k + d_k1 - 2.0 * s_k
  dk8 = d_k + 1e-8
  h1 = s8 - d_k
  a1 = 2.0 * h1

  u = rw
  v = -rw * x_k
  u2 = u * u
  uv2 = 2.0 * u * v
  v2 = v * v
  q2 = -(mid * u2)
  q1 = mid * u - mid * uv2
  q0 = mid * v - mid * v2 + s8
  a2c = h1 * u2
  a1c = h1 * uv2 + dk8 * u
  a0c = h1 * v2 + dk8 * v
  s82 = s8 * s8
  lane5 = io == 5
  zero5 = lambda t: jnp.where(lane5, 0.0, t)
  one5 = lambda t: jnp.where(lane5, 1.0, t)
  t_q2[...] = zero5(q2)
  t_q1[...] = zero5(q1)
  t_q0[...] = one5(q0)
  t_p2[...] = zero5(y_k * q2 + dy * a2c)
  t_p1[...] = one5(y_k * q1 + dy * a1c)
  t_p0[...] = zero5(y_k * q0 + dy * a0c)
  t_g2[...] = zero5((mid * u2) * s82)
  t_g1[...] = zero5((mid * uv2 + a1 * u) * s82)
  t_g0[...] = one5((mid * v2 + a1 * v + dk8) * s82)
  t_khi[...] = jnp.where(io == 4, jnp.float32(_TB),
                         jnp.where(io >= 5, jnp.float32(3.4e38), x_k1))

  k1 = jnp.sum(jnp.where(io == 0, x_k1, 0.0))
  k2 = jnp.sum(jnp.where(io == 1, x_k1, 0.0))
  k3 = jnp.sum(jnp.where(io == 2, x_k1, 0.0))
  k4 = jnp.sum(jnp.where(io == 3, x_k1, 0.0))
  iof = io.astype(jnp.float32)
  for j in range(_NLUT // 16):
    lo = (iof + (16.0 * j - 1.0)) * (1.0 / _CELL_SCALE) - _TB
    bj = (jnp.where(k1 < lo, 1, 0) + jnp.where(k2 < lo, 1, 0)
          + jnp.where(k3 < lo, 1, 0) + jnp.where(k4 < lo, 1, 0))
    if j == 0:
      bj = jnp.where(io == 0, 5, bj)
    lut[pl.ds(16 * j, 16)] = bj

  def compute(xb, zb, ljb):
    @plsc.parallel_loop(0, _CH, step=_LANES, unroll=4)
    def _loop(off):
      sl = pl.ds(off, _LANES)
      xv = xb[sl]
      uf = xv * _CELL_SCALE + (0.5 * _NCELL + 1.0)
      uf = jnp.minimum(jnp.maximum(uf, 0.0), float(_NCELL + 1))
      ui = uf.astype(jnp.int32)
      b0 = plsc.load_gather(lut, [ui])
      khi = plsc.load_gather(t_khi, [b0])
      b = b0 + jnp.where(khi < xv, 1, 0)
      g_q2 = plsc.load_gather(t_q2, [b])
      g_q1 = plsc.load_gather(t_q1, [b])
      g_q0 = plsc.load_gather(t_q0, [b])
      g_p2 = plsc.load_gather(t_p2, [b])
      g_p1 = plsc.load_gather(t_p1, [b])
      g_p0 = plsc.load_gather(t_p0, [b])
      g_g2 = plsc.load_gather(t_g2, [b])
      g_g1 = plsc.load_gather(t_g1, [b])
      g_g0 = plsc.load_gather(t_g0, [b])

      qx = (g_q2 * xv + g_q1) * xv + g_q0
      px = (g_p2 * xv + g_p1) * xv + g_p0
      gx = (g_g2 * xv + g_g1) * xv + g_g0
      inv = 1.0 / qx
      zb[sl] = px * inv
      ljb[sl] = _vlog(gx * (inv * inv))

  out_d = [None, None]
  for g in range(_CHUNKS):
    b = g % 2
    off = base + g * _CH
    in_d[b].wait()
    if g + 1 < _CHUNKS:
      nb = (g + 1) % 2
      in_d[nb] = pltpu.async_copy(x_hbm.at[pl.ds(off + _CH, _CH)],
                                  xbufs[nb], sems_in[nb])
    if out_d[b] is not None:
      out_d[b][0].wait()
      out_d[b][1].wait()
    compute(xbufs[b], zbufs[b], ljbufs[b])
    out_d[b] = (
        pltpu.async_copy(zbufs[b], z_hbm.at[pl.ds(off, _CH)], sems_out[b]),
        pltpu.async_copy(ljbufs[b], lj_hbm.at[pl.ds(off, _CH)],
                         sems_out[b]),
    )
  out_d[0][0].wait()
  out_d[0][1].wait()
  out_d[1][0].wait()
  out_d[1][1].wait()


@jax.jit
def _run(x_flat, params):
  mesh = plsc.VectorSubcoreMesh(core_axis_name="c", subcore_axis_name="s",
                                num_cores=_NC, num_subcores=_NS)
  f = pl.kernel(
      _sc_body,
      out_type=[jax.ShapeDtypeStruct((_N,), jnp.float32),
                jax.ShapeDtypeStruct((_N,), jnp.float32)],
      mesh=mesh,
      compiler_params=pltpu.CompilerParams(needs_layout_passes=False),
      scratch_types=[
          pltpu.VMEM((16,), jnp.float32),
          pltpu.VMEM((16,), jnp.float32),
          pltpu.VMEM((16,), jnp.float32),
          pltpu.VMEM((16,), jnp.float32),
          pltpu.VMEM((16,), jnp.float32),
          pltpu.VMEM((16,), jnp.float32),
          pltpu.VMEM((16,), jnp.float32),
          pltpu.VMEM((16,), jnp.float32),
          pltpu.VMEM((16,), jnp.float32),
          pltpu.VMEM((16,), jnp.float32),
          pltpu.VMEM((16,), jnp.float32),
          pltpu.VMEM((_NLUT,), jnp.int32),
          pltpu.VMEM((_CH,), jnp.float32),
          pltpu.VMEM((_CH,), jnp.float32),
          pltpu.VMEM((_CH,), jnp.float32),
          pltpu.VMEM((_CH,), jnp.float32),
          pltpu.VMEM((_CH,), jnp.float32),
          pltpu.VMEM((_CH,), jnp.float32),
          pltpu.SemaphoreType.DMA,
          pltpu.SemaphoreType.DMA,
          pltpu.SemaphoreType.DMA,
          pltpu.SemaphoreType.DMA,
      ],
  )
  return f(x_flat, params)


def kernel(x, params):
  z, lj = _run(x[:, 0], params)
  return (z[:, None], lj)


# speedup vs baseline: 1.7923x; 1.0182x over previous
import functools

import jax
import jax.numpy as jnp
from jax import lax
from jax.experimental import pallas as pl
from jax.experimental.pallas import tpu as pltpu
from jax.experimental.pallas import tpu_sc as plsc

_NUM_BINS = 5
_TB = 2.5
_LN2 = 0.6931471805599453
_MAGIC = 0x3F3504F3
_C1 = 1.9999695786510276
_C3 = 0.6769402206514328

_NC = 2
_NS = 16
_NW = _NC * _NS
_LANES = 16

_N = 4194304
_PER_W = _N // _NW
_CH = 16384
_CHUNKS = _PER_W // _CH

_CELL_SCALE = 819.2
_CELL_OFF = 2868.0
_LUT_LO = 820
_LUT_HI = 4916
_LUT_MAX = 4917.0
_NLUT = 4928
_LUT_INV = 5.0 / 4096.0
_LUT_X0 = 3.5009765625


def _vlog(t):
  bits = plsc.bitcast(t, jnp.int32)
  e = (bits - _MAGIC) >> 23
  m = plsc.bitcast(bits - (e << 23), jnp.float32)
  s = (m - 1.0) / (m + 1.0)
  return e.astype(jnp.float32) * _LN2 + s * (_C1 + _C3 * (s * s))


def _lane_shift(v, idx):
  return v.at[idx].get(mode="promise_in_bounds")


def _sc_body(x_hbm, p_hbm, z_hbm, lj_hbm, pbuf, t_q2, t_q1, t_q0, t_p2, t_p1,
             t_p0, t_g2, t_g1, t_g0, lut, xbuf0, xbuf1, zbuf0, zbuf1,
             ljbuf0, ljbuf1, sem_in0, sem_in1, sem_out0, sem_out1):
  wid = lax.axis_index("s") * _NC + lax.axis_index("c")
  base = wid * _PER_W
  xbufs = (xbuf0, xbuf1)
  zbufs = (zbuf0, zbuf1)
  ljbufs = (ljbuf0, ljbuf1)
  sems_in = (sem_in0, sem_in1)
  sems_out = (sem_out0, sem_out1)

  in_d = [None, None]
  in_d[0] = pltpu.async_copy(x_hbm.at[pl.ds(base, _CH)], xbufs[0],
                             sems_in[0])

  pltpu.sync_copy(p_hbm, pbuf)
  pv = pbuf[...]
  io = lax.iota(jnp.int32, 16)
  mask_w = io < _NUM_BINS
  mask_h = (io >= _NUM_BINS) & (io < 2 * _NUM_BINS)
  neg = jnp.float32(-3.4e38)

  mw = jnp.max(jnp.where(mask_w, pv, neg))
  ew = jnp.exp(pv - mw)
  sw = jnp.sum(jnp.where(mask_w, ew, 0.0))
  w_v = (ew * (2.0 * _TB)) / sw
  mh = jnp.max(jnp.where(mask_h, pv, neg))
  eh = jnp.exp(pv - mh)
  sh = jnp.sum(jnp.where(mask_h, eh, 0.0))
  h_v = (eh * (2.0 * _TB)) / sh
  d_v = jnp.maximum(pv, 0.0) + _vlog(1.0 + jnp.exp(-jnp.abs(pv))) + 1e-5

  cw = plsc.cumsum(jnp.where(mask_w, w_v, 0.0))
  ch = plsc.cumsum(jnp.where(mask_h, h_v, 0.0))

  cap = jnp.int32(15)
  x_k1 = cw - _TB
  x_k = jnp.where(io == 0, -_TB,
                  _lane_shift(cw, jnp.maximum(io - 1, 0)) - _TB)
  rw = 1.0 / (x_k1 - x_k + 1e-8)
  y_k = jnp.where(io == 0, -_TB,
                  _lane_shift(ch, jnp.minimum(io + 4, cap)) - _TB)
  y_k1 = _lane_shift(ch, jnp.minimum(io + 5, cap)) - _TB
  dy = y_k1 - y_k
  d_k = _lane_shift(d_v, jnp.minimum(io + 10, cap))
  d_k1 = _lane_shift(d_v, jnp.minimum(io + 11, cap))
  s_k = _lane_shift(h_v, jnp.minimum(io + 5, cap)) / w_v
  s8 = s_k + 1e-8
  mid = d_k + d_k1 - 2.0 * s_k
  dk8 = d_k + 1e-8
  h1 = s8 - d_k
  a1 = 2.0 * h1

  u = rw
  v = -rw * x_k
  u2 = u * u
  uv2 = 2.0 * u * v
  v2 = v * v
  q2 = -(mid * u2)
  q1 = mid * u - mid * uv2
  q0 = mid * v - mid * v2 + s8
  a2c = h1 * u2
  a1c = h1 * uv2 + dk8 * u
  a0c = h1 * v2 + dk8 * v
  s82 = s8 * s8
  lane5 = io == 5
  zero5 = lambda t: jnp.where(lane5, 0.0, t)
  one5 = lambda t: jnp.where(lane5, 1.0, t)
  t_q2[...] = zero5(q2)
  t_q1[...] = zero5(q1)
  t_q0[...] = one5(q0)
  t_p2[...] = zero5(y_k * q2 + dy * a2c)
  t_p1[...] = one5(y_k * q1 + dy * a1c)
  t_p0[...] = zero5(y_k * q0 + dy * a0c)
  t_g2[...] = zero5((mid * u2) * s82)
  t_g1[...] = zero5((mid * uv2 + a1 * u) * s82)
  t_g0[...] = one5((mid * v2 + a1 * v + dk8) * s82)

  k1 = jnp.sum(jnp.where(io == 0, x_k1, 0.0))
  k2 = jnp.sum(jnp.where(io == 1, x_k1, 0.0))
  k3 = jnp.sum(jnp.where(io == 2, x_k1, 0.0))
  k4 = jnp.sum(jnp.where(io == 3, x_k1, 0.0))
  iof = io.astype(jnp.float32)

  def build_lut(j, _):
    cf = (io + j * 16).astype(jnp.float32)
    lo = cf * _LUT_INV - _LUT_X0
    bj = (jnp.where(k1 < lo, 1, 0) + jnp.where(k2 < lo, 1, 0)
          + jnp.where(k3 < lo, 1, 0) + jnp.where(k4 < lo, 1, 0))
    tail = (cf < float(_LUT_LO)) | (cf >= float(_LUT_HI))
    lut[pl.ds(j * 16, 16)] = jnp.where(tail, 5, bj)
    return 0

  lax.fori_loop(0, _NLUT // 16, build_lut, 0)

  def compute(xb, zb, ljb):
    @plsc.parallel_loop(0, _CH, step=_LANES, unroll=4)
    def _loop(off):
      sl = pl.ds(off, _LANES)
      xv = xb[sl]
      uf = xv * _CELL_SCALE + _CELL_OFF
      uf = jnp.minimum(jnp.maximum(uf, 0.0), _LUT_MAX)
      ui = uf.astype(jnp.int32)
      b = plsc.load_gather(lut, [ui])
      g_q2 = plsc.load_gather(t_q2, [b])
      g_q1 = plsc.load_gather(t_q1, [b])
      g_q0 = plsc.load_gather(t_q0, [b])
      g_p2 = plsc.load_gather(t_p2, [b])
      g_p1 = plsc.load_gather(t_p1, [b])
      g_p0 = plsc.load_gather(t_p0, [b])
      g_g2 = plsc.load_gather(t_g2, [b])
      g_g1 = plsc.load_gather(t_g1, [b])
      g_g0 = plsc.load_gather(t_g0, [b])

      qx = (g_q2 * xv + g_q1) * xv + g_q0
      px = (g_p2 * xv + g_p1) * xv + g_p0
      gx = (g_g2 * xv + g_g1) * xv + g_g0
      inv = 1.0 / qx
      zb[sl] = px * inv
      ljb[sl] = _vlog(gx * (inv * inv))

  out_d = [None, None]
  for g in range(_CHUNKS):
    b = g % 2
    off = base + g * _CH
    in_d[b].wait()
    if g + 1 < _CHUNKS:
      nb = (g + 1) % 2
      in_d[nb] = pltpu.async_copy(x_hbm.at[pl.ds(off + _CH, _CH)],
                                  xbufs[nb], sems_in[nb])
    if out_d[b] is not None:
      out_d[b][0].wait()
      out_d[b][1].wait()
    compute(xbufs[b], zbufs[b], ljbufs[b])
    out_d[b] = (
        pltpu.async_copy(zbufs[b], z_hbm.at[pl.ds(off, _CH)], sems_out[b]),
        pltpu.async_copy(ljbufs[b], lj_hbm.at[pl.ds(off, _CH)],
                         sems_out[b]),
    )
  out_d[0][0].wait()
  out_d[0][1].wait()
  out_d[1][0].wait()
  out_d[1][1].wait()


@jax.jit
def _run(x_flat, params):
  mesh = plsc.VectorSubcoreMesh(core_axis_name="c", subcore_axis_name="s",
                                num_cores=_NC, num_subcores=_NS)
  f = pl.kernel(
      _sc_body,
      out_type=[jax.ShapeDtypeStruct((_N,), jnp.float32),
                jax.ShapeDtypeStruct((_N,), jnp.float32)],
      mesh=mesh,
      compiler_params=pltpu.CompilerParams(needs_layout_passes=False),
      scratch_types=[
          pltpu.VMEM((16,), jnp.float32),
          pltpu.VMEM((16,), jnp.float32),
          pltpu.VMEM((16,), jnp.float32),
          pltpu.VMEM((16,), jnp.float32),
          pltpu.VMEM((16,), jnp.float32),
          pltpu.VMEM((16,), jnp.float32),
          pltpu.VMEM((16,), jnp.float32),
          pltpu.VMEM((16,), jnp.float32),
          pltpu.VMEM((16,), jnp.float32),
          pltpu.VMEM((16,), jnp.float32),
          pltpu.VMEM((_NLUT,), jnp.int32),
          pltpu.VMEM((_CH,), jnp.float32),
          pltpu.VMEM((_CH,), jnp.float32),
          pltpu.VMEM((_CH,), jnp.float32),
          pltpu.VMEM((_CH,), jnp.float32),
          pltpu.VMEM((_CH,), jnp.float32),
          pltpu.VMEM((_CH,), jnp.float32),
          pltpu.SemaphoreType.DMA,
          pltpu.SemaphoreType.DMA,
          pltpu.SemaphoreType.DMA,
          pltpu.SemaphoreType.DMA,
      ],
  )
  return f(x_flat, params)


def kernel(x, params):
  z, lj = _run(x[:, 0], params)
  return (z[:, None], lj)
